# Initial kernel scaffold; baseline (speedup 1.0000x reference)
import jax
import jax.numpy as jnp
from jax.experimental import pallas as pl


def kernel(x, Wg, bg, Wqkv, Wproj, bproj):
    raise NotImplementedError("write your pallas kernel here")


# speedup vs baseline: 5.3440x; 5.3440x over previous
import jax
import jax.numpy as jnp
from jax.experimental import pallas as pl


def _fold_kernel(wqkv_ref, weff_ref):
    w = wqkv_ref[0]
    d = w.shape[0]
    weff_ref[0] = w[:, :d] + w[:, d:2 * d] + w[:, 2 * d:]


def _main_kernel(x_ref, wg_ref, bg_ref, weff_ref, wproj_ref, bproj_ref, o_ref):
    xb = x_ref[...]
    blk, dim = xb.shape
    n_exp = wg_ref.shape[1]
    heads = 4
    dh = dim // heads
    scale = dh ** -0.5

    scores = jnp.dot(xb, wg_ref[...], preferred_element_type=jnp.float32)
    scores = scores + bg_ref[...]
    m = jnp.max(scores, axis=1, keepdims=True)
    ex = jnp.exp(scores - m)
    p = ex / jnp.sum(ex, axis=1, keepdims=True)

    lane = jax.lax.broadcasted_iota(jnp.int32, p.shape, 1)
    i1 = jnp.argmax(p, axis=1)[:, None]
    m1 = lane == i1
    v1 = jnp.max(p, axis=1, keepdims=True)
    p2 = jnp.where(m1, -1e30, p)
    i2 = jnp.argmax(p2, axis=1)[:, None]
    m2 = lane == i2
    v2 = jnp.max(p2, axis=1, keepdims=True)
    w = jnp.where(m1, v1, 0.0) + jnp.where(m2, v2, 0.0)

    s = jnp.zeros((blk, dim), jnp.float32)
    for e in range(n_exp):
        se = jnp.dot(xb, weff_ref[e], preferred_element_type=jnp.float32)
        s = s + w[:, e:e + 1] * se

    sh = [s[:, j * dh:(j + 1) * dh] for j in range(heads)]
    outs = []
    for i in range(heads):
        lg = jnp.concatenate(
            [jnp.sum(sh[i] * sh[j], axis=1, keepdims=True) * scale
             for j in range(heads)], axis=1)
        mx = jnp.max(lg, axis=1, keepdims=True)
        el = jnp.exp(lg - mx)
        pr = el / jnp.sum(el, axis=1, keepdims=True)
        acc = pr[:, 0:1] * sh[0]
        for j in range(1, heads):
            acc = acc + pr[:, j:j + 1] * sh[j]
        outs.append(acc)
    attn_out = jnp.concatenate(outs, axis=1)

    o_ref[...] = (jnp.dot(attn_out, wproj_ref[...],
                          preferred_element_type=jnp.float32)
                  + bproj_ref[...])


def kernel(x, Wg, bg, Wqkv, Wproj, bproj):
    b, dim = x.shape
    n_exp = Wg.shape[1]
    heads = 4
    dh = dim // heads
    blk = 512

    wproj_perm = Wproj.reshape(dh, heads, dim).swapaxes(0, 1).reshape(dim, dim)

    weff = pl.pallas_call(
        _fold_kernel,
        grid=(n_exp,),
        in_specs=[pl.BlockSpec((1, dim, 3 * dim), lambda e: (e, 0, 0))],
        out_specs=pl.BlockSpec((1, dim, dim), lambda e: (e, 0, 0)),
        out_shape=jax.ShapeDtypeStruct((n_exp, dim, dim), jnp.float32),
    )(Wqkv)

    out = pl.pallas_call(
        _main_kernel,
        grid=(b // blk,),
        in_specs=[
            pl.BlockSpec((blk, dim), lambda i: (i, 0)),
            pl.BlockSpec((dim, n_exp), lambda i: (0, 0)),
            pl.BlockSpec((1, n_exp), lambda i: (0, 0)),
            pl.BlockSpec((n_exp, dim, dim), lambda i: (0, 0, 0)),
            pl.BlockSpec((dim, dim), lambda i: (0, 0)),
            pl.BlockSpec((1, dim), lambda i: (0, 0)),
        ],
        out_specs=pl.BlockSpec((blk, dim), lambda i: (i, 0)),
        out_shape=jax.ShapeDtypeStruct((b, dim), jnp.float32),
    )(x, Wg, bg.reshape(1, n_exp), weff, wproj_perm, bproj.reshape(1, dim))
    return out


# speedup vs baseline: 5.8700x; 1.0984x over previous
import jax
import jax.numpy as jnp
from jax.experimental import pallas as pl


def _fold_kernel(wqkv_ref, weff_ref):
    w = wqkv_ref[0]
    d = w.shape[0]
    weff_ref[0] = (w[:, :d] + w[:, d:2 * d] + w[:, 2 * d:]).astype(jnp.bfloat16)


def _main_kernel(x_ref, wg_ref, bg_ref, weff_ref, wproj_ref, bproj_ref, o_ref):
    xb = x_ref[...]
    blk, dim = xb.shape
    n_exp = wg_ref.shape[1]
    heads = 4
    dh = dim // heads
    scale = dh ** -0.5

    scores = jnp.dot(xb, wg_ref[...], preferred_element_type=jnp.float32)
    scores = scores + bg_ref[...]
    m = jnp.max(scores, axis=1, keepdims=True)
    ex = jnp.exp(scores - m)
    p = ex / jnp.sum(ex, axis=1, keepdims=True)

    lane = jax.lax.broadcasted_iota(jnp.int32, p.shape, 1)
    i1 = jnp.argmax(p, axis=1)[:, None]
    m1 = lane == i1
    v1 = jnp.max(p, axis=1, keepdims=True)
    p2 = jnp.where(m1, -1e30, p)
    i2 = jnp.argmax(p2, axis=1)[:, None]
    m2 = lane == i2
    v2 = jnp.max(p2, axis=1, keepdims=True)
    w = jnp.where(m1, v1, 0.0) + jnp.where(m2, v2, 0.0)

    xb16 = xb.astype(jnp.bfloat16)
    s = jnp.zeros((blk, dim), jnp.float32)
    for e in range(n_exp):
        se = jnp.dot(xb16, weff_ref[e], preferred_element_type=jnp.float32)
        s = s + w[:, e:e + 1] * se

    sh = [s[:, j * dh:(j + 1) * dh] for j in range(heads)]
    outs = []
    for i in range(heads):
        lg = jnp.concatenate(
            [jnp.sum(sh[i] * sh[j], axis=1, keepdims=True) * scale
             for j in range(heads)], axis=1)
        mx = jnp.max(lg, axis=1, keepdims=True)
        el = jnp.exp(lg - mx)
        pr = el / jnp.sum(el, axis=1, keepdims=True)
        acc = pr[:, 0:1] * sh[0]
        for j in range(1, heads):
            acc = acc + pr[:, j:j + 1] * sh[j]
        outs.append(acc)
    attn_out = jnp.concatenate(outs, axis=1)

    o_ref[...] = (jnp.dot(attn_out.astype(jnp.bfloat16), wproj_ref[...],
                          preferred_element_type=jnp.float32)
                  + bproj_ref[...])


def kernel(x, Wg, bg, Wqkv, Wproj, bproj):
    b, dim = x.shape
    n_exp = Wg.shape[1]
    heads = 4
    dh = dim // heads
    blk = 512

    wproj_perm = Wproj.reshape(dh, heads, dim).swapaxes(0, 1).reshape(dim, dim)

    weff = pl.pallas_call(
        _fold_kernel,
        grid=(n_exp,),
        in_specs=[pl.BlockSpec((1, dim, 3 * dim), lambda e: (e, 0, 0))],
        out_specs=pl.BlockSpec((1, dim, dim), lambda e: (e, 0, 0)),
        out_shape=jax.ShapeDtypeStruct((n_exp, dim, dim), jnp.bfloat16),
    )(Wqkv)

    out = pl.pallas_call(
        _main_kernel,
        grid=(b // blk,),
        in_specs=[
            pl.BlockSpec((blk, dim), lambda i: (i, 0)),
            pl.BlockSpec((dim, n_exp), lambda i: (0, 0)),
            pl.BlockSpec((1, n_exp), lambda i: (0, 0)),
            pl.BlockSpec((n_exp, dim, dim), lambda i: (0, 0, 0)),
            pl.BlockSpec((dim, dim), lambda i: (0, 0)),
            pl.BlockSpec((1, dim), lambda i: (0, 0)),
        ],
        out_specs=pl.BlockSpec((blk, dim), lambda i: (i, 0)),
        out_shape=jax.ShapeDtypeStruct((b, dim), jnp.float32),
    )(x, Wg, bg.reshape(1, n_exp), weff, wproj_perm.astype(jnp.bfloat16),
      bproj.reshape(1, dim))
    return out
